# baseline (speedup 1.0000x reference)
import functools

import jax
import jax.numpy as jnp
from jax import lax
from jax.experimental import pallas as pl
from jax.experimental.pallas import tpu as pltpu
from jax.experimental.pallas import tpu_sc as plsc

N = 320000
D = 128
NUM_SEG = 10000

NC = 2
NS = 16
NW = NC * NS

CHUNK = 128
GRP = 16
GRPS_PER_CHUNK = CHUNK // GRP

RT = 512
NBLK = N // RT
KT = 224
S = (NBLK - KT) * RT
SOFF = NBLK - KT
NCHUNKS = S // CHUNK
WIN = 128
ACC_ROWS = NUM_SEG + WIN
STRIPE = 624
STRIPE_LAST = NUM_SEG - (NS - 1) * STRIPE


def _sc_segment_partials(x, batch2d, zeros_stripe):
    mesh = plsc.VectorSubcoreMesh(core_axis_name="c", subcore_axis_name="s")

    @functools.partial(
        pl.kernel,
        mesh=mesh,
        out_type=jax.ShapeDtypeStruct((NC, NUM_SEG, D), jnp.float32),
        scratch_types=[
            pltpu.VMEM((CHUNK, D), jnp.float32),
            pltpu.VMEM((CHUNK, D), jnp.float32),
            pltpu.VMEM((GRPS_PER_CHUNK, GRP), jnp.int32),
            pltpu.VMEM((GRPS_PER_CHUNK, GRP), jnp.int32),
            pltpu.VMEM_SHARED((NUM_SEG, D), jnp.float32),
            pltpu.SemaphoreType.DMA,
            pltpu.SemaphoreType.DMA,
            pltpu.SemaphoreType.DMA,
        ],
    )
    def k(x_hbm, b_hbm, z_hbm, out_hbm, rows0, rows1, idx0, idx1, acc,
          sem0, sem1, sem_sc):
        cid = lax.axis_index("c")
        sid = lax.axis_index("s")
        wid = sid * NC + cid

        @pl.when(sid < NS - 1)
        def _():
            pltpu.sync_copy(z_hbm.at[pl.ds(0, STRIPE)],
                            acc.at[pl.ds(sid * STRIPE, STRIPE)])

        @pl.when(sid == NS - 1)
        def _():
            pltpu.sync_copy(z_hbm,
                            acc.at[pl.ds((NS - 1) * STRIPE, STRIPE_LAST)])

        plsc.subcore_barrier()

        nchunks_w = (NCHUNKS - wid + NW - 1) // NW
        slots = ((rows0, idx0, sem0), (rows1, idx1, sem1))

        def fill(slot, i):
            rows_v, idx_v, sem = slot
            c = wid + i * NW
            pltpu.async_copy(x_hbm.at[pl.ds(c * CHUNK, CHUNK)], rows_v, sem)
            pltpu.async_copy(
                b_hbm.at[pl.ds(c * GRPS_PER_CHUNK, GRPS_PER_CHUNK)], idx_v, sem)

        def wait_fill(slot, i):
            rows_v, idx_v, sem = slot
            c = wid + i * NW
            pltpu.make_async_copy(
                x_hbm.at[pl.ds(c * CHUNK, CHUNK)], rows_v, sem).wait()
            pltpu.make_async_copy(
                b_hbm.at[pl.ds(c * GRPS_PER_CHUNK, GRPS_PER_CHUNK)], idx_v,
                sem).wait()

        def scatter(slot):
            rows_v, idx_v, _ = slot
            hs = [
                pltpu.async_copy(
                    rows_v.at[pl.ds(j * GRP, GRP)],
                    acc.at[idx_v.at[j]],
                    sem_sc,
                    add=True,
                )
                for j in range(GRPS_PER_CHUNK)
            ]
            for h in hs:
                h.wait()

        fill(slots[0], 0)
        fill(slots[1], 1)

        def body(p, carry):
            for b in (0, 1):
                i = 2 * p + b

                @pl.when(i < nchunks_w)
                def _():
                    wait_fill(slots[b], i)
                    scatter(slots[b])

                    @pl.when(i + 2 < nchunks_w)
                    def _():
                        fill(slots[b], i + 2)

            return carry

        lax.fori_loop(0, (nchunks_w + 1) // 2, body, 0)
        plsc.subcore_barrier()

        @pl.when(sid < NS - 1)
        def _():
            pltpu.sync_copy(
                acc.at[pl.ds(sid * STRIPE, STRIPE)],
                out_hbm.at[cid].at[pl.ds(sid * STRIPE, STRIPE)],
            )

        @pl.when(sid == NS - 1)
        def _():
            pltpu.sync_copy(
                acc.at[pl.ds((NS - 1) * STRIPE, STRIPE_LAST)],
                out_hbm.at[cid].at[pl.ds((NS - 1) * STRIPE, STRIPE_LAST)],
            )

    return k(x, batch2d, zeros_stripe)


def _tc_segment_partial(x, batch3d, lob):
    def body(lo_sref, idx_ref, x_ref, out_ref, acc_ref):
        i = pl.program_id(0)

        @pl.when(i == 0)
        def _():
            acc_ref[...] = jnp.zeros((ACC_ROWS, D), jnp.float32)

        lo = lo_sref[SOFF + i]
        hi = lo_sref[SOFF + i + 1]
        base0 = (lo // 8) * 8
        nw = (hi - base0) // WIN + 1
        idxv = idx_ref[0]
        xblk = x_ref[...]
        x_hi = xblk.astype(jnp.bfloat16)
        x_lo = (xblk - x_hi.astype(jnp.float32)).astype(jnp.bfloat16)

        def win(w, carry):
            base = base0 + w * WIN
            rows = jax.lax.broadcasted_iota(jnp.int32, (WIN, RT), 0) + base
            oh = (rows == idxv).astype(jnp.bfloat16)
            dims = (((1,), (0,)), ((), ()))
            contrib = (
                jax.lax.dot_general(
                    oh, x_hi, dims, preferred_element_type=jnp.float32)
                + jax.lax.dot_general(
                    oh, x_lo, dims, preferred_element_type=jnp.float32))
            acc_ref[pl.ds(base, WIN), :] += contrib
            return carry

        lax.fori_loop(0, nw, win, 0)

        @pl.when(i == KT - 1)
        def _():
            out_ref[...] = acc_ref[0:NUM_SEG, :]

    grid_spec = pltpu.PrefetchScalarGridSpec(
        num_scalar_prefetch=1,
        grid=(KT,),
        in_specs=[
            pl.BlockSpec((1, 1, RT), lambda i, s: (SOFF + i, 0, 0)),
            pl.BlockSpec((RT, D), lambda i, s: (SOFF + i, 0)),
        ],
        out_specs=pl.BlockSpec((NUM_SEG, D), lambda i, s: (0, 0)),
        scratch_shapes=[pltpu.VMEM((ACC_ROWS, D), jnp.float32)],
    )
    return pl.pallas_call(
        body,
        grid_spec=grid_spec,
        out_shape=jax.ShapeDtypeStruct((NUM_SEG, D), jnp.float32),
    )(lob, batch3d, x)


def _add_partials(partials, ptc):
    def body(a_ref, b_ref, c_ref, o_ref):
        o_ref[...] = a_ref[0] + b_ref[0] + c_ref[...]

    blk = 1000
    return pl.pallas_call(
        body,
        grid=(NUM_SEG // blk,),
        in_specs=[
            pl.BlockSpec((1, blk, D), lambda i: (0, i, 0)),
            pl.BlockSpec((1, blk, D), lambda i: (1, i, 0)),
            pl.BlockSpec((blk, D), lambda i: (i, 0)),
        ],
        out_specs=pl.BlockSpec((blk, D), lambda i: (i, 0)),
        out_shape=jax.ShapeDtypeStruct((NUM_SEG, D), jnp.float32),
    )(partials, partials, ptc)


@jax.jit
def kernel(x, batch):
    batch32 = batch.astype(jnp.int32)
    batch2d = batch32.reshape(N // GRP, GRP)
    batch3d = batch32.reshape(NBLK, 1, RT)
    lob = jnp.concatenate(
        [batch32[::RT], jnp.array([NUM_SEG - 1], jnp.int32)])
    zeros_stripe = jnp.zeros((STRIPE_LAST, D), jnp.float32)
    ptc = _tc_segment_partial(x, batch3d, lob)
    partials = _sc_segment_partials(x, batch2d, zeros_stripe)
    return _add_partials(partials, ptc)


# speedup vs baseline: 1.3364x; 1.3364x over previous
import functools

import jax
import jax.numpy as jnp
from jax import lax
from jax.experimental import pallas as pl
from jax.experimental.pallas import tpu as pltpu
from jax.experimental.pallas import tpu_sc as plsc

N = 320000
D = 128
NUM_SEG = 10000

NC = 2
NS = 16
NW = NC * NS

CHUNK = 128
GRP = 16
GRPS_PER_CHUNK = CHUNK // GRP
NCHUNKS = N // CHUNK
STRIPE = 624
STRIPE_LAST = NUM_SEG - (NS - 1) * STRIPE


def _sc_segment_partials(x, batch2d, zeros_stripe):
    mesh = plsc.VectorSubcoreMesh(core_axis_name="c", subcore_axis_name="s")

    @functools.partial(
        pl.kernel,
        mesh=mesh,
        out_type=jax.ShapeDtypeStruct((NC, NUM_SEG, D), jnp.float32),
        scratch_types=[
            pltpu.VMEM((CHUNK, D), jnp.float32),
            pltpu.VMEM((CHUNK, D), jnp.float32),
            pltpu.VMEM((GRPS_PER_CHUNK, GRP), jnp.int32),
            pltpu.VMEM((GRPS_PER_CHUNK, GRP), jnp.int32),
            pltpu.VMEM_SHARED((NUM_SEG, D), jnp.float32),
            pltpu.SemaphoreType.DMA,
            pltpu.SemaphoreType.DMA,
            pltpu.SemaphoreType.DMA,
        ],
    )
    def k(x_hbm, b_hbm, z_hbm, out_hbm, rows0, rows1, idx0, idx1, acc,
          sem0, sem1, sem_sc):
        cid = lax.axis_index("c")
        sid = lax.axis_index("s")
        wid = sid * NC + cid

        def prime(slot, i):
            rows_v, idx_v, sem = slot
            c = wid + i * NW
            pltpu.async_copy(x_hbm.at[pl.ds(c * CHUNK, CHUNK)], rows_v, sem)
            pltpu.async_copy(
                b_hbm.at[pl.ds(c * GRPS_PER_CHUNK, GRPS_PER_CHUNK)], idx_v, sem)

        prime((rows0, idx0, sem0), 0)
        prime((rows1, idx1, sem1), 1)

        @pl.when(sid < NS - 1)
        def _():
            pltpu.sync_copy(z_hbm.at[pl.ds(0, STRIPE)],
                            acc.at[pl.ds(sid * STRIPE, STRIPE)])

        @pl.when(sid == NS - 1)
        def _():
            pltpu.sync_copy(z_hbm,
                            acc.at[pl.ds((NS - 1) * STRIPE, STRIPE_LAST)])

        plsc.subcore_barrier()

        nchunks_w = (NCHUNKS - wid + NW - 1) // NW
        slots = ((rows0, idx0, sem0), (rows1, idx1, sem1))

        def fill(slot, i):
            rows_v, idx_v, sem = slot
            c = wid + i * NW
            pltpu.async_copy(x_hbm.at[pl.ds(c * CHUNK, CHUNK)], rows_v, sem)
            pltpu.async_copy(
                b_hbm.at[pl.ds(c * GRPS_PER_CHUNK, GRPS_PER_CHUNK)], idx_v, sem)

        def wait_fill(slot, i):
            rows_v, idx_v, sem = slot
            c = wid + i * NW
            pltpu.make_async_copy(
                x_hbm.at[pl.ds(c * CHUNK, CHUNK)], rows_v, sem).wait()
            pltpu.make_async_copy(
                b_hbm.at[pl.ds(c * GRPS_PER_CHUNK, GRPS_PER_CHUNK)], idx_v,
                sem).wait()

        def scatter(slot):
            rows_v, idx_v, _ = slot
            hs = [
                pltpu.async_copy(
                    rows_v.at[pl.ds(j * GRP, GRP)],
                    acc.at[idx_v.at[j]],
                    sem_sc,
                    add=True,
                )
                for j in range(GRPS_PER_CHUNK)
            ]
            for h in hs:
                h.wait()

        def body(p, carry):
            for b in (0, 1):
                i = 2 * p + b

                @pl.when(i < nchunks_w)
                def _():
                    wait_fill(slots[b], i)
                    scatter(slots[b])

                    @pl.when(i + 2 < nchunks_w)
                    def _():
                        fill(slots[b], i + 2)

            return carry

        lax.fori_loop(0, (nchunks_w + 1) // 2, body, 0)
        plsc.subcore_barrier()

        @pl.when(sid < NS - 1)
        def _():
            pltpu.sync_copy(
                acc.at[pl.ds(sid * STRIPE, STRIPE)],
                out_hbm.at[cid].at[pl.ds(sid * STRIPE, STRIPE)],
            )

        @pl.when(sid == NS - 1)
        def _():
            pltpu.sync_copy(
                acc.at[pl.ds((NS - 1) * STRIPE, STRIPE_LAST)],
                out_hbm.at[cid].at[pl.ds((NS - 1) * STRIPE, STRIPE_LAST)],
            )

    return k(x, batch2d, zeros_stripe)


def _add_partials(partials):
    def body(a_ref, b_ref, o_ref):
        o_ref[...] = a_ref[0] + b_ref[0]

    blk = 1000
    return pl.pallas_call(
        body,
        grid=(NUM_SEG // blk,),
        in_specs=[
            pl.BlockSpec((1, blk, D), lambda i: (0, i, 0)),
            pl.BlockSpec((1, blk, D), lambda i: (1, i, 0)),
        ],
        out_specs=pl.BlockSpec((blk, D), lambda i: (i, 0)),
        out_shape=jax.ShapeDtypeStruct((NUM_SEG, D), jnp.float32),
    )(partials, partials)


@jax.jit
def kernel(x, batch):
    batch2d = batch.astype(jnp.int32).reshape(N // GRP, GRP)
    zeros_stripe = jnp.zeros((STRIPE_LAST, D), jnp.float32)
    partials = _sc_segment_partials(x, batch2d, zeros_stripe)
    return _add_partials(partials)


# speedup vs baseline: 1.4035x; 1.0502x over previous
import functools

import jax
import jax.numpy as jnp
from jax import lax
from jax.experimental import pallas as pl
from jax.experimental.pallas import tpu as pltpu
from jax.experimental.pallas import tpu_sc as plsc

N = 320000
D = 128
NUM_SEG = 10000

NC = 2
NS = 16
NW = NC * NS

CHUNK = 128
NCHUNKS = N // CHUNK
NOCT = NCHUNKS // 8
OCT_LO = NOCT // NW
OCT_HI_WORKERS = NOCT - OCT_LO * NW
TAIL_BASE = NOCT * 8
TAIL_N = NCHUNKS - TAIL_BASE
STRIPE = 624
STRIPE_LAST = NUM_SEG - (NS - 1) * STRIPE


def _sc_segment_partials(x, batch2d, zeros_stripe):
    mesh = plsc.VectorSubcoreMesh(core_axis_name="c", subcore_axis_name="s")

    @functools.partial(
        pl.kernel,
        mesh=mesh,
        out_type=jax.ShapeDtypeStruct((NC, NUM_SEG, D), jnp.float32),
        scratch_types=[
            pltpu.VMEM((CHUNK, D), jnp.float32),
            pltpu.VMEM((CHUNK, D), jnp.float32),
            pltpu.VMEM((8, CHUNK), jnp.int32),
            pltpu.VMEM((8, CHUNK), jnp.int32),
            pltpu.VMEM_SHARED((NUM_SEG, D), jnp.float32),
            pltpu.SemaphoreType.DMA,
            pltpu.SemaphoreType.DMA,
            pltpu.SemaphoreType.DMA,
            pltpu.SemaphoreType.DMA,
        ],
    )
    def k(x_hbm, b_hbm, z_hbm, out_hbm, rows0, rows1, idx0, idx1, acc,
          sem0, sem1, sem_i, sem_sc):
        cid = lax.axis_index("c")
        sid = lax.axis_index("s")
        wid = sid * NC + cid

        base_oct = wid * OCT_LO + jnp.minimum(wid, OCT_HI_WORKERS)
        n_oct = OCT_LO + (wid < OCT_HI_WORKERS).astype(jnp.int32)
        base = base_oct * 8
        count = n_oct * 8
        rows = (rows0, rows1)
        rsem = (sem0, sem1)
        idxb = (idx0, idx1)

        def fill(b, c):
            pltpu.async_copy(x_hbm.at[pl.ds(c * CHUNK, CHUNK)], rows[b],
                             rsem[b])

        def wait_fill(b, c):
            pltpu.make_async_copy(x_hbm.at[pl.ds(c * CHUNK, CHUNK)], rows[b],
                                  rsem[b]).wait()

        fill(0, base)
        fill(1, base + 1)
        pltpu.async_copy(b_hbm.at[pl.ds(base_oct * 8, 8)], idx0, sem_i)
        pltpu.async_copy(b_hbm.at[pl.ds((base_oct + 1) * 8, 8)], idx1, sem_i)

        @pl.when(sid < NS - 1)
        def _():
            pltpu.sync_copy(z_hbm.at[pl.ds(0, STRIPE)],
                            acc.at[pl.ds(sid * STRIPE, STRIPE)])

        @pl.when(sid == NS - 1)
        def _():
            pltpu.sync_copy(z_hbm,
                            acc.at[pl.ds((NS - 1) * STRIPE, STRIPE_LAST)])

        plsc.subcore_barrier()

        def octad(o, carry):
            ob = idxb[0], idxb[1]
            for ib_static in (0, 1):
                @pl.when(o % 2 == ib_static)
                def _():
                    ib = ob[ib_static]
                    pltpu.make_async_copy(
                        b_hbm.at[pl.ds((base_oct + o) * 8, 8)], ib,
                        sem_i).wait()

                    for kk in range(8):
                        c = base + o * 8 + kk
                        b = kk % 2
                        wait_fill(b, c)
                        pltpu.async_copy(rows[b], acc.at[ib.at[kk]], sem_sc,
                                         add=True).wait()

                        @pl.when(c + 2 < base + count)
                        def _():
                            fill(b, c + 2)

                    @pl.when(o + 2 < n_oct)
                    def _():
                        pltpu.async_copy(
                            b_hbm.at[pl.ds((base_oct + o + 2) * 8, 8)], ib,
                            sem_i)

            return carry

        lax.fori_loop(0, n_oct, octad, 0)

        @pl.when(wid == NW - 1)
        def _():
            pltpu.sync_copy(b_hbm.at[pl.ds(TAIL_BASE, TAIL_N)],
                            idx0.at[pl.ds(0, TAIL_N)])
            for kk in range(TAIL_N):
                pltpu.sync_copy(
                    x_hbm.at[pl.ds((TAIL_BASE + kk) * CHUNK, CHUNK)], rows0)
                pltpu.sync_copy(rows0, acc.at[idx0.at[kk]], add=True)

        plsc.subcore_barrier()

        @pl.when(sid < NS - 1)
        def _():
            pltpu.sync_copy(
                acc.at[pl.ds(sid * STRIPE, STRIPE)],
                out_hbm.at[cid].at[pl.ds(sid * STRIPE, STRIPE)],
            )

        @pl.when(sid == NS - 1)
        def _():
            pltpu.sync_copy(
                acc.at[pl.ds((NS - 1) * STRIPE, STRIPE_LAST)],
                out_hbm.at[cid].at[pl.ds((NS - 1) * STRIPE, STRIPE_LAST)],
            )

    return k(x, batch2d, zeros_stripe)


def _add_partials(partials):
    def body(a_ref, b_ref, o_ref):
        o_ref[...] = a_ref[0] + b_ref[0]

    blk = 1000
    return pl.pallas_call(
        body,
        grid=(NUM_SEG // blk,),
        in_specs=[
            pl.BlockSpec((1, blk, D), lambda i: (0, i, 0)),
            pl.BlockSpec((1, blk, D), lambda i: (1, i, 0)),
        ],
        out_specs=pl.BlockSpec((blk, D), lambda i: (i, 0)),
        out_shape=jax.ShapeDtypeStruct((NUM_SEG, D), jnp.float32),
    )(partials, partials)


@jax.jit
def kernel(x, batch):
    batch2d = batch.astype(jnp.int32).reshape(NCHUNKS, CHUNK)
    zeros_stripe = jnp.zeros((STRIPE_LAST, D), jnp.float32)
    partials = _sc_segment_partials(x, batch2d, zeros_stripe)
    return _add_partials(partials)


# speedup vs baseline: 1.4220x; 1.0132x over previous
import functools

import jax
import jax.numpy as jnp
from jax import lax
from jax.experimental import pallas as pl
from jax.experimental.pallas import tpu as pltpu
from jax.experimental.pallas import tpu_sc as plsc

N = 320000
D = 128
NUM_SEG = 10000

NC = 2
NS = 16
NW = NC * NS

CHUNK = 128
NCHUNKS = N // CHUNK
NOCT = NCHUNKS // 8
OCT_LO = NOCT // NW
OCT_HI_WORKERS = NOCT - OCT_LO * NW
TAIL_BASE = NOCT * 8
TAIL_N = NCHUNKS - TAIL_BASE
STRIPE = 624
STRIPE_LAST = NUM_SEG - (NS - 1) * STRIPE


def _sc_segment_partials(x, batch2d, zeros_stripe):
    mesh = plsc.VectorSubcoreMesh(core_axis_name="c", subcore_axis_name="s")

    @functools.partial(
        pl.kernel,
        mesh=mesh,
        out_type=jax.ShapeDtypeStruct((NC, NUM_SEG, D), jnp.float32),
        scratch_types=[
            pltpu.VMEM((CHUNK, D), jnp.float32),
            pltpu.VMEM((CHUNK, D), jnp.float32),
            pltpu.VMEM((8, CHUNK), jnp.int32),
            pltpu.VMEM((8, CHUNK), jnp.int32),
            pltpu.VMEM_SHARED((NUM_SEG, D), jnp.float32),
            pltpu.SemaphoreType.DMA,
            pltpu.SemaphoreType.DMA,
            pltpu.SemaphoreType.DMA,
            pltpu.SemaphoreType.DMA,
        ],
    )
    def k(x_hbm, b_hbm, z_hbm, out_hbm, rows0, rows1, idx0, idx1, acc,
          sem0, sem1, sem_i, sem_sc):
        cid = lax.axis_index("c")
        sid = lax.axis_index("s")
        wid = sid * NC + cid

        base_oct = wid * OCT_LO + jnp.minimum(wid, OCT_HI_WORKERS)
        n_oct = OCT_LO + (wid < OCT_HI_WORKERS).astype(jnp.int32)
        base = base_oct * 8
        count = n_oct * 8
        rows = (rows0, rows1)
        rsem = (sem0, sem1)
        idxb = (idx0, idx1)

        def fill(b, c):
            pltpu.async_copy(x_hbm.at[pl.ds(c * CHUNK, CHUNK)], rows[b],
                             rsem[b])

        def wait_fill(b, c):
            pltpu.make_async_copy(x_hbm.at[pl.ds(c * CHUNK, CHUNK)], rows[b],
                                  rsem[b]).wait()

        fill(0, base)
        fill(1, base + 1)
        pltpu.async_copy(b_hbm.at[pl.ds(base_oct * 8, 8)], idx0, sem_i)
        pltpu.async_copy(b_hbm.at[pl.ds((base_oct + 1) * 8, 8)], idx1, sem_i)

        @pl.when(sid < NS - 1)
        def _():
            pltpu.sync_copy(z_hbm.at[pl.ds(0, STRIPE)],
                            acc.at[pl.ds(sid * STRIPE, STRIPE)])

        @pl.when(sid == NS - 1)
        def _():
            pltpu.sync_copy(z_hbm,
                            acc.at[pl.ds((NS - 1) * STRIPE, STRIPE_LAST)])

        plsc.subcore_barrier()

        def octad(o, carry):
            ob = idxb[0], idxb[1]
            for ib_static in (0, 1):
                @pl.when(o % 2 == ib_static)
                def _():
                    ib = ob[ib_static]
                    pltpu.make_async_copy(
                        b_hbm.at[pl.ds((base_oct + o) * 8, 8)], ib,
                        sem_i).wait()

                    for kk in range(8):
                        c = base + o * 8 + kk
                        b = kk % 2
                        wait_fill(b, c)
                        pltpu.async_copy(rows[b], acc.at[ib.at[kk]], sem_sc,
                                         add=True).wait()

                        @pl.when(c + 2 < base + count)
                        def _():
                            fill(b, c + 2)

                    @pl.when(o + 2 < n_oct)
                    def _():
                        pltpu.async_copy(
                            b_hbm.at[pl.ds((base_oct + o + 2) * 8, 8)], ib,
                            sem_i)

            return carry

        lax.fori_loop(0, n_oct, octad, 0)

        @pl.when(wid == NW - 1)
        def _():
            pltpu.sync_copy(b_hbm.at[pl.ds(TAIL_BASE, TAIL_N)],
                            idx0.at[pl.ds(0, TAIL_N)])
            for kk in range(TAIL_N):
                pltpu.sync_copy(
                    x_hbm.at[pl.ds((TAIL_BASE + kk) * CHUNK, CHUNK)], rows0)
                pltpu.sync_copy(rows0, acc.at[idx0.at[kk]], add=True)

        plsc.subcore_barrier()

        @pl.when(sid < NS - 1)
        def _():
            pltpu.sync_copy(
                acc.at[pl.ds(sid * STRIPE, STRIPE)],
                out_hbm.at[cid].at[pl.ds(sid * STRIPE, STRIPE)],
            )

        @pl.when(sid == NS - 1)
        def _():
            pltpu.sync_copy(
                acc.at[pl.ds((NS - 1) * STRIPE, STRIPE_LAST)],
                out_hbm.at[cid].at[pl.ds((NS - 1) * STRIPE, STRIPE_LAST)],
            )

    return k(x, batch2d, zeros_stripe)


def _add_partials(partials):
    def body(a_ref, b_ref, o_ref):
        o_ref[...] = a_ref[0] + b_ref[0]

    blk = 2000
    return pl.pallas_call(
        body,
        grid=(NUM_SEG // blk,),
        in_specs=[
            pl.BlockSpec((1, blk, D), lambda i: (0, i, 0)),
            pl.BlockSpec((1, blk, D), lambda i: (1, i, 0)),
        ],
        out_specs=pl.BlockSpec((blk, D), lambda i: (i, 0)),
        out_shape=jax.ShapeDtypeStruct((NUM_SEG, D), jnp.float32),
    )(partials, partials)


@jax.jit
def kernel(x, batch):
    batch2d = batch.astype(jnp.int32).reshape(NCHUNKS, CHUNK)
    zeros_stripe = jnp.zeros((STRIPE_LAST, D), jnp.float32)
    partials = _sc_segment_partials(x, batch2d, zeros_stripe)
    return _add_partials(partials)
